# baseline (speedup 1.0000x reference)
import jax
import jax.numpy as jnp
from jax.experimental import pallas as pl
from jax.experimental.pallas import tpu as pltpu

_R = 4608
_C = 1024
_BR = 512


def _tc_body(sal_ref, fix_ref, out_ref, acc_ref):
    i = pl.program_id(0)
    ni = pl.num_programs(0)

    @pl.when(i == 0)
    def _init():
        acc_ref[0] = 0.0
        acc_ref[1] = 0.0
        acc_ref[2] = 0.0
        acc_ref[3] = 0.0

    s = sal_ref[...]
    f = fix_ref[...]
    m = f > 0.1
    acc_ref[0] += jnp.sum(s)
    acc_ref[1] += jnp.sum(s * s)
    acc_ref[2] += jnp.sum(jnp.where(m, s, 0.0))
    acc_ref[3] += jnp.sum(m.astype(jnp.float32))

    @pl.when(i == ni - 1)
    def _fin():
        s1 = acc_ref[0]
        s2 = acc_ref[1]
        ms = acc_ref[2]
        cnt = acc_ref[3]
        n = jnp.float32(_R * _C)
        mean = s1 / n
        var = (s2 - s1 * s1 / n) / (n - 1.0)
        std = jnp.sqrt(var)
        out_ref[0] = (ms - cnt * mean) / (std * cnt)


def kernel(sal_map, fix):
    sal2 = sal_map.reshape(_R, _C)
    fix2 = fix.reshape(_R, _C)
    out = pl.pallas_call(
        _tc_body,
        grid=(_R // _BR,),
        in_specs=[
            pl.BlockSpec((_BR, _C), lambda i: (i, 0)),
            pl.BlockSpec((_BR, _C), lambda i: (i, 0)),
        ],
        out_specs=pl.BlockSpec(memory_space=pltpu.SMEM),
        out_shape=jax.ShapeDtypeStruct((1,), jnp.float32),
        scratch_shapes=[pltpu.SMEM((4,), jnp.float32)],
    )(sal2, fix2)
    return out[0]


# speedup vs baseline: 1.0103x; 1.0103x over previous
import jax
import jax.numpy as jnp
from jax.experimental import pallas as pl
from jax.experimental.pallas import tpu as pltpu

_R = 4608
_C = 1024
_BR = 512


def _tc_body(sal_ref, fix_ref, out_ref, acc_ref):
    i = pl.program_id(0)
    ni = pl.num_programs(0)

    @pl.when(i == 0)
    def _init():
        acc_ref[...] = jnp.zeros_like(acc_ref)

    s = sal_ref[...]
    f = fix_ref[...]
    m = f > 0.1
    sb = s.reshape(_BR // 8, 8, _C)
    fb = jnp.where(m, s, 0.0).reshape(_BR // 8, 8, _C)
    cb = m.astype(jnp.float32).reshape(_BR // 8, 8, _C)
    acc_ref[0] += jnp.sum(sb, axis=0)
    acc_ref[1] += jnp.sum(sb * sb, axis=0)
    acc_ref[2] += jnp.sum(fb, axis=0)
    acc_ref[3] += jnp.sum(cb, axis=0)

    @pl.when(i == ni - 1)
    def _fin():
        s1 = jnp.sum(acc_ref[0])
        s2 = jnp.sum(acc_ref[1])
        ms = jnp.sum(acc_ref[2])
        cnt = jnp.sum(acc_ref[3])
        n = jnp.float32(_R * _C)
        mean = s1 / n
        var = (s2 - s1 * s1 / n) / (n - 1.0)
        std = jnp.sqrt(var)
        out_ref[0] = (ms - cnt * mean) / (std * cnt)


def kernel(sal_map, fix):
    sal2 = sal_map.reshape(_R, _C)
    fix2 = fix.reshape(_R, _C)
    out = pl.pallas_call(
        _tc_body,
        grid=(_R // _BR,),
        in_specs=[
            pl.BlockSpec((_BR, _C), lambda i: (i, 0)),
            pl.BlockSpec((_BR, _C), lambda i: (i, 0)),
        ],
        out_specs=pl.BlockSpec(memory_space=pltpu.SMEM),
        out_shape=jax.ShapeDtypeStruct((1,), jnp.float32),
        scratch_shapes=[pltpu.VMEM((4, 8, _C), jnp.float32)],
    )(sal2, fix2)
    return out[0]


# speedup vs baseline: 3.7190x; 3.6813x over previous
import jax
import jax.numpy as jnp
from jax.experimental import pallas as pl
from jax.experimental.pallas import tpu as pltpu

_B = 32
_H = 384
_W = 384
_BB = 4


def _tc_body(sal_ref, fix_ref, out_ref, acc_ref):
    i = pl.program_id(0)
    ni = pl.num_programs(0)

    @pl.when(i == 0)
    def _init():
        acc_ref[...] = jnp.zeros_like(acc_ref)

    s = sal_ref[...]
    f = fix_ref[...]
    m = f > 0.1
    r = _BB * _H // 8
    sb = s.reshape(r, 8, _W)
    fb = jnp.where(m, s, 0.0).reshape(r, 8, _W)
    cb = m.astype(jnp.float32).reshape(r, 8, _W)
    acc_ref[0] += jnp.sum(sb, axis=0)
    acc_ref[1] += jnp.sum(sb * sb, axis=0)
    acc_ref[2] += jnp.sum(fb, axis=0)
    acc_ref[3] += jnp.sum(cb, axis=0)

    @pl.when(i == ni - 1)
    def _fin():
        s1 = jnp.sum(acc_ref[0])
        s2 = jnp.sum(acc_ref[1])
        ms = jnp.sum(acc_ref[2])
        cnt = jnp.sum(acc_ref[3])
        n = jnp.float32(_B * _H * _W)
        mean = s1 / n
        var = (s2 - s1 * s1 / n) / (n - 1.0)
        std = jnp.sqrt(var)
        out_ref[0] = (ms - cnt * mean) / (std * cnt)


def kernel(sal_map, fix):
    out = pl.pallas_call(
        _tc_body,
        grid=(_B // _BB,),
        in_specs=[
            pl.BlockSpec((_BB, 1, _H, _W), lambda i: (i, 0, 0, 0)),
            pl.BlockSpec((_BB, 1, _H, _W), lambda i: (i, 0, 0, 0)),
        ],
        out_specs=pl.BlockSpec(memory_space=pltpu.SMEM),
        out_shape=jax.ShapeDtypeStruct((1,), jnp.float32),
        scratch_shapes=[pltpu.VMEM((4, 8, _W), jnp.float32)],
    )(sal_map, fix)
    return out[0]
